# baseline (speedup 1.0000x reference)
import functools

import jax
import jax.numpy as jnp
from jax import lax
from jax.experimental import pallas as pl
from jax.experimental.pallas import tpu as pltpu
from jax.experimental.pallas import tpu_sc as plsc

N = 10000
E = 320000
D = 128

NC = 2
NS = 16
NW = NC * NS

E_W = E // NW
CHS = 125
NCHS = E_W // CHS
GRP = 8

N_PAD = 10240
DEG_WIN = 16

_mesh = plsc.VectorSubcoreMesh(
    core_axis_name="c", subcore_axis_name="s", num_cores=NC, num_subcores=NS
)


def _fill_1d(ref, n, value):
    v = jnp.full((16,), value, jnp.float32)

    def body(i, _):
        ref[pl.ds(i * 16, 16)] = v
        return 0

    lax.fori_loop(0, n // 16, body, 0)


@functools.partial(
    pl.kernel,
    out_type=jax.ShapeDtypeStruct((NC, N_PAD), jnp.float32),
    mesh=_mesh,
    scratch_types=[
        pltpu.VMEM((NCHS, CHS), jnp.int32),
        pltpu.VMEM((128,), jnp.float32),
        pltpu.VMEM((640,), jnp.float32),
        pltpu.VMEM_SHARED((N_PAD,), jnp.float32),
        pltpu.SemaphoreType.DMA,
    ],
)
def _deg(rows_hbm, out_hbm, idx2, ones_v, zeros_v, deg_sh, sem):
    cid = lax.axis_index("c")
    sid = lax.axis_index("s")
    wid = sid * NC + cid

    _fill_1d(ones_v, 128, 1.0)
    _fill_1d(zeros_v, 640, 0.0)
    pltpu.sync_copy(zeros_v, deg_sh.at[pl.ds(sid * 640, 640)])
    pltpu.sync_copy(rows_hbm.at[wid], idx2)
    plsc.subcore_barrier()

    ones_src = ones_v.at[pl.ds(0, CHS)]

    def prime(j, _):
        pltpu.async_copy(ones_src, deg_sh.at[idx2.at[j]], sem, add=True)
        return 0

    lax.fori_loop(0, DEG_WIN, prime, 0)

    def step(j, _):
        pltpu.make_async_copy(ones_src, deg_sh.at[idx2.at[j]], sem).wait()

        @pl.when(j < NCHS - DEG_WIN)
        def _():
            pltpu.async_copy(ones_src, deg_sh.at[idx2.at[j + DEG_WIN]], sem,
                             add=True)

        return 0

    lax.fori_loop(0, NCHS, step, 0)

    plsc.subcore_barrier()

    @pl.when(sid == 0)
    def _():
        pltpu.sync_copy(deg_sh, out_hbm.at[cid])


@functools.partial(
    pl.kernel,
    out_type=jax.ShapeDtypeStruct((NC, N, D), jnp.float32),
    mesh=_mesh,
    scratch_types=[
        pltpu.VMEM((NCHS, CHS), jnp.int32),
        [pltpu.VMEM((GRP, CHS), jnp.int32)] * 2,
        [pltpu.VMEM((CHS, D), jnp.float32)] * 2,
        [pltpu.SemaphoreType.DMA] * 2,
        [pltpu.SemaphoreType.DMA] * 2,
        [pltpu.SemaphoreType.DMA] * 2,
        pltpu.SemaphoreType.DMA,
        pltpu.VMEM_SHARED((N, D), jnp.float32),
    ],
)
def _spmm(rows_hbm, cols_hbm, g_hbm, acc_hbm,
          colv2, rowbs, gbufs, gsems, ssems, rsems, isem, acc_sh):
    cid = lax.axis_index("c")
    sid = lax.axis_index("s")
    wid = sid * NC + cid
    ngrp = NCHS // GRP

    def rows_src(g):
        return rows_hbm.at[wid, pl.ds(pl.multiple_of(g * GRP, GRP), GRP)]

    c_idx = pltpu.async_copy(cols_hbm.at[wid], colv2, isem)
    for p in range(2):
        pltpu.async_copy(rows_src(p), rowbs[p], rsems[p])

    zb = gbufs[0]

    def zrow(i, _):
        def zcol(j, _):
            zb[i, pl.ds(j * 16, 16)] = jnp.zeros((16,), jnp.float32)
            return 0
        lax.fori_loop(0, D // 16, zcol, 0)
        return 0

    lax.fori_loop(0, CHS, zrow, 0)
    r0 = sid * (N // NS)
    for k in range(5):
        pltpu.sync_copy(zb, acc_sh.at[pl.ds(r0 + k * CHS, CHS)])
    c_idx.wait()
    plsc.subcore_barrier()

    pltpu.async_copy(g_hbm.at[colv2.at[0]], gbufs[0], gsems[0])

    def slot(j, i, b, p):
        pltpu.make_async_copy(g_hbm.at[colv2.at[j]], gbufs[b],
                              gsems[b]).wait()
        pltpu.async_copy(gbufs[b], acc_sh.at[rowbs[p].at[i]], ssems[b],
                         add=True)

        @pl.when(j >= 1)
        def _():
            pltpu.make_async_copy(gbufs[1 - b], acc_sh.at[rowbs[p].at[i]],
                                  ssems[1 - b]).wait()

        @pl.when(j < NCHS - 1)
        def _():
            pltpu.async_copy(g_hbm.at[colv2.at[j + 1]], gbufs[1 - b],
                             gsems[1 - b])

    def group(g, p):
        pltpu.make_async_copy(rows_src(g), rowbs[p], rsems[p]).wait()
        for i in range(GRP):
            slot(g * GRP + i, i, i % 2, p)

        @pl.when(g < ngrp - 2)
        def _():
            pltpu.async_copy(rows_src(g + 2), rowbs[p], rsems[p])

    def pair(kk, _):
        group(2 * kk, 0)
        group(2 * kk + 1, 1)
        return 0

    lax.fori_loop(0, ngrp // 2, pair, 0)

    pltpu.make_async_copy(gbufs[1], acc_sh.at[rowbs[1].at[GRP - 1]],
                          ssems[1]).wait()

    plsc.subcore_barrier()

    @pl.when(sid < NS - 1)
    def _():
        s0 = sid * 624
        pltpu.sync_copy(acc_sh.at[pl.ds(s0, 624)],
                        acc_hbm.at[cid, pl.ds(s0, 624)])

    @pl.when(sid == NS - 1)
    def _():
        pltpu.sync_copy(acc_sh.at[pl.ds(624 * (NS - 1), 640)],
                        acc_hbm.at[cid, pl.ds(624 * (NS - 1), 640)])


_BLK = 1000


def _matmul_body(x_ref, wt_ref, b_ref, h_ref):
    h_ref[...] = jnp.dot(x_ref[...], wt_ref[...],
                         preferred_element_type=jnp.float32) + b_ref[...]


def _scale_body(h_ref, deg_ref, g_ref):
    deg = jnp.sum(deg_ref[...], axis=1, keepdims=True) + 1.0
    g_ref[...] = h_ref[...] * lax.rsqrt(deg)


def _combine_body(acc_ref, g_ref, deg_ref, o_ref):
    deg = jnp.sum(deg_ref[...], axis=1, keepdims=True) + 1.0
    o_ref[...] = (acc_ref[0] + acc_ref[1] + g_ref[...]) * lax.rsqrt(deg)


def kernel(x, edge_index, W, b):
    ei = edge_index.astype(jnp.int32)
    rows3 = ei[0].reshape(NW, NCHS, CHS)
    cols3 = ei[1].reshape(NW, NCHS, CHS)

    h = pl.pallas_call(
        _matmul_body,
        grid=(N // _BLK,),
        in_specs=[
            pl.BlockSpec((_BLK, D), lambda i: (i, 0)),
            pl.BlockSpec((D, D), lambda i: (0, 0)),
            pl.BlockSpec((1, D), lambda i: (0, 0)),
        ],
        out_specs=pl.BlockSpec((_BLK, D), lambda i: (i, 0)),
        out_shape=jax.ShapeDtypeStruct((N, D), jnp.float32),
    )(x, W.T, b.reshape(1, D))

    degs = _deg(rows3)
    degs_t = degs[:, :N].T

    g = pl.pallas_call(
        _scale_body,
        grid=(N // _BLK,),
        in_specs=[
            pl.BlockSpec((_BLK, D), lambda i: (i, 0)),
            pl.BlockSpec((_BLK, 2), lambda i: (i, 0)),
        ],
        out_specs=pl.BlockSpec((_BLK, D), lambda i: (i, 0)),
        out_shape=jax.ShapeDtypeStruct((N, D), jnp.float32),
    )(h, degs_t)

    accs = _spmm(rows3, cols3, g)

    out = pl.pallas_call(
        _combine_body,
        grid=(N // _BLK,),
        in_specs=[
            pl.BlockSpec((NC, _BLK, D), lambda i: (0, i, 0)),
            pl.BlockSpec((_BLK, D), lambda i: (i, 0)),
            pl.BlockSpec((_BLK, 2), lambda i: (i, 0)),
        ],
        out_specs=pl.BlockSpec((_BLK, D), lambda i: (i, 0)),
        out_shape=jax.ShapeDtypeStruct((N, D), jnp.float32),
    )(accs, g, degs_t)
    return out


# speedup vs baseline: 1.2019x; 1.2019x over previous
import functools

import jax
import jax.numpy as jnp
from jax import lax
from jax.experimental import pallas as pl
from jax.experimental.pallas import tpu as pltpu
from jax.experimental.pallas import tpu_sc as plsc

N = 10000
E = 320000
D = 128

NC = 2
NS = 16
NW = NC * NS

E_W = E // NW
CHS = 125
NCHS = E_W // CHS
GRP = 8

N_PAD = 10240
DEG_WIN = 16

_mesh = plsc.VectorSubcoreMesh(
    core_axis_name="c", subcore_axis_name="s", num_cores=NC, num_subcores=NS
)


def _fill_1d(ref, n, value):
    v = jnp.full((16,), value, jnp.float32)

    def body(i, _):
        ref[pl.ds(i * 16, 16)] = v
        return 0

    lax.fori_loop(0, n // 16, body, 0)


@functools.partial(
    pl.kernel,
    out_type=jax.ShapeDtypeStruct((NC, N_PAD), jnp.float32),
    mesh=_mesh,
    scratch_types=[
        pltpu.VMEM((NCHS, CHS), jnp.int32),
        pltpu.VMEM((128,), jnp.float32),
        pltpu.VMEM((640,), jnp.float32),
        pltpu.VMEM_SHARED((N_PAD,), jnp.float32),
        pltpu.SemaphoreType.DMA,
    ],
)
def _deg(rows_hbm, out_hbm, idx2, ones_v, zeros_v, deg_sh, sem):
    cid = lax.axis_index("c")
    sid = lax.axis_index("s")
    wid = sid * NC + cid

    _fill_1d(ones_v, 128, 1.0)
    _fill_1d(zeros_v, 640, 0.0)
    pltpu.sync_copy(zeros_v, deg_sh.at[pl.ds(sid * 640, 640)])
    pltpu.sync_copy(rows_hbm.at[wid], idx2)
    plsc.subcore_barrier()

    ones_src = ones_v.at[pl.ds(0, CHS)]

    def prime(j, _):
        pltpu.async_copy(ones_src, deg_sh.at[idx2.at[j]], sem, add=True)
        return 0

    lax.fori_loop(0, DEG_WIN, prime, 0)

    def step(j, _):
        pltpu.make_async_copy(ones_src, deg_sh.at[idx2.at[j]], sem).wait()

        @pl.when(j < NCHS - DEG_WIN)
        def _():
            pltpu.async_copy(ones_src, deg_sh.at[idx2.at[j + DEG_WIN]], sem,
                             add=True)

        return 0

    lax.fori_loop(0, NCHS, step, 0)

    plsc.subcore_barrier()

    @pl.when(sid == 0)
    def _():
        pltpu.sync_copy(deg_sh, out_hbm.at[cid])


@functools.partial(
    pl.kernel,
    out_type=jax.ShapeDtypeStruct((NC, N, D), jnp.float32),
    mesh=_mesh,
    scratch_types=[
        pltpu.VMEM((NCHS, CHS), jnp.int32),
        [pltpu.VMEM((GRP, CHS), jnp.int32)] * 2,
        [pltpu.VMEM((CHS, D), jnp.float32)] * 2,
        [pltpu.SemaphoreType.DMA] * 2,
        [pltpu.SemaphoreType.DMA] * 2,
        [pltpu.SemaphoreType.DMA] * 2,
        pltpu.SemaphoreType.DMA,
        pltpu.VMEM_SHARED((N, D), jnp.float32),
    ],
)
def _spmm(rows_hbm, cols_hbm, g_hbm, acc_hbm,
          colv2, rowbs, gbufs, gsems, ssems, rsems, isem, acc_sh):
    cid = lax.axis_index("c")
    sid = lax.axis_index("s")
    wid = sid * NC + cid
    ngrp = NCHS // GRP

    def rows_src(g):
        return rows_hbm.at[wid, pl.ds(pl.multiple_of(g * GRP, GRP), GRP)]

    c_idx = pltpu.async_copy(cols_hbm.at[wid], colv2, isem)
    for p in range(2):
        pltpu.async_copy(rows_src(p), rowbs[p], rsems[p])

    r0 = sid * (N // NS)

    @pl.when(cid == 0)
    def _():
        @pl.when(sid < NS - 1)
        def _():
            s0 = sid * 624
            pltpu.sync_copy(g_hbm.at[pl.ds(s0, 624)],
                            acc_sh.at[pl.ds(s0, 624)])

        @pl.when(sid == NS - 1)
        def _():
            pltpu.sync_copy(g_hbm.at[pl.ds(624 * (NS - 1), 640)],
                            acc_sh.at[pl.ds(624 * (NS - 1), 640)])

    @pl.when(cid == 1)
    def _():
        zb = gbufs[0]

        def zrow(i, _):
            def zcol(j, _):
                zb[i, pl.ds(j * 16, 16)] = jnp.zeros((16,), jnp.float32)
                return 0
            lax.fori_loop(0, D // 16, zcol, 0)
            return 0

        lax.fori_loop(0, CHS, zrow, 0)
        for k in range(5):
            pltpu.sync_copy(zb, acc_sh.at[pl.ds(r0 + k * CHS, CHS)])

    c_idx.wait()
    plsc.subcore_barrier()

    for b in range(2):
        pltpu.async_copy(g_hbm.at[colv2.at[b]], gbufs[b], gsems[b])

    def group(g, p):
        pltpu.make_async_copy(rows_src(g), rowbs[p], rsems[p]).wait()
        for i in range(GRP):
            b = i % 2
            j = g * GRP + i
            pltpu.make_async_copy(g_hbm.at[colv2.at[j]], gbufs[b],
                                  gsems[b]).wait()
            pltpu.sync_copy(gbufs[b], acc_sh.at[rowbs[p].at[i]], add=True)

            @pl.when(j < NCHS - 2)
            def _(j=j, b=b):
                pltpu.async_copy(g_hbm.at[colv2.at[j + 2]], gbufs[b],
                                 gsems[b])

        @pl.when(g < ngrp - 2)
        def _():
            pltpu.async_copy(rows_src(g + 2), rowbs[p], rsems[p])

    def pair(kk, _):
        group(2 * kk, 0)
        group(2 * kk + 1, 1)
        return 0

    lax.fori_loop(0, ngrp // 2, pair, 0)

    plsc.subcore_barrier()

    @pl.when(sid < NS - 1)
    def _():
        s0 = sid * 624
        pltpu.sync_copy(acc_sh.at[pl.ds(s0, 624)],
                        acc_hbm.at[cid, pl.ds(s0, 624)])

    @pl.when(sid == NS - 1)
    def _():
        pltpu.sync_copy(acc_sh.at[pl.ds(624 * (NS - 1), 640)],
                        acc_hbm.at[cid, pl.ds(624 * (NS - 1), 640)])


_BLK = 2000


def _linear_body(x_ref, wt_ref, b_ref, deg_ref, g_ref):
    h = jnp.dot(x_ref[...], wt_ref[...], preferred_element_type=jnp.float32)
    h = h + b_ref[...]
    deg = jnp.sum(deg_ref[...], axis=1, keepdims=True) + 1.0
    g_ref[...] = h * lax.rsqrt(deg)


def _combine_body(acc_ref, deg_ref, o_ref):
    deg = jnp.sum(deg_ref[...], axis=1, keepdims=True) + 1.0
    o_ref[...] = (acc_ref[0] + acc_ref[1]) * lax.rsqrt(deg)


def kernel(x, edge_index, W, b):
    ei = edge_index.astype(jnp.int32)
    rows3 = ei[0].reshape(NW, NCHS, CHS)
    cols3 = ei[1].reshape(NW, NCHS, CHS)

    degs = _deg(rows3)
    degs_t = degs[:, :N].T

    g = pl.pallas_call(
        _linear_body,
        grid=(N // _BLK,),
        in_specs=[
            pl.BlockSpec((_BLK, D), lambda i: (i, 0)),
            pl.BlockSpec((D, D), lambda i: (0, 0)),
            pl.BlockSpec((1, D), lambda i: (0, 0)),
            pl.BlockSpec((_BLK, 2), lambda i: (i, 0)),
        ],
        out_specs=pl.BlockSpec((_BLK, D), lambda i: (i, 0)),
        out_shape=jax.ShapeDtypeStruct((N, D), jnp.float32),
    )(x, W.T, b.reshape(1, D), degs_t)

    accs = _spmm(rows3, cols3, g)

    out = pl.pallas_call(
        _combine_body,
        grid=(N // _BLK,),
        in_specs=[
            pl.BlockSpec((NC, _BLK, D), lambda i: (0, i, 0)),
            pl.BlockSpec((_BLK, 2), lambda i: (i, 0)),
        ],
        out_specs=pl.BlockSpec((_BLK, D), lambda i: (i, 0)),
        out_shape=jax.ShapeDtypeStruct((N, D), jnp.float32),
    )(accs, degs_t)
    return out


# speedup vs baseline: 1.2102x; 1.0070x over previous
import functools

import jax
import jax.numpy as jnp
from jax import lax
from jax.experimental import pallas as pl
from jax.experimental.pallas import tpu as pltpu
from jax.experimental.pallas import tpu_sc as plsc

N = 10000
E = 320000
D = 128

NC = 2
NS = 16
NW = NC * NS

CHS = 128
NCHS = 80
E_PAD = NW * NCHS * CHS
E_W = E_PAD // NW
GRP = 8

N_PAD = 10240
N_ACC = 10080
DEG_WIN = 16

_mesh = plsc.VectorSubcoreMesh(
    core_axis_name="c", subcore_axis_name="s", num_cores=NC, num_subcores=NS
)


def _fill_1d(ref, n, value):
    v = jnp.full((16,), value, jnp.float32)

    def body(i, _):
        ref[pl.ds(i * 16, 16)] = v
        return 0

    lax.fori_loop(0, n // 16, body, 0)


@functools.partial(
    pl.kernel,
    out_type=jax.ShapeDtypeStruct((NC, N_PAD), jnp.float32),
    mesh=_mesh,
    scratch_types=[
        pltpu.VMEM((NCHS, CHS), jnp.int32),
        pltpu.VMEM((128,), jnp.float32),
        pltpu.VMEM((640,), jnp.float32),
        pltpu.VMEM_SHARED((N_PAD,), jnp.float32),
        pltpu.SemaphoreType.DMA,
    ],
)
def _deg(rows_hbm, out_hbm, idx2, ones_v, zeros_v, deg_sh, sem):
    cid = lax.axis_index("c")
    sid = lax.axis_index("s")
    wid = sid * NC + cid

    _fill_1d(ones_v, 128, 1.0)
    _fill_1d(zeros_v, 640, 0.0)
    pltpu.sync_copy(zeros_v, deg_sh.at[pl.ds(sid * 640, 640)])
    pltpu.sync_copy(rows_hbm.at[wid], idx2)
    plsc.subcore_barrier()

    ones_src = ones_v.at[pl.ds(0, CHS)]

    def prime(j, _):
        pltpu.async_copy(ones_src, deg_sh.at[idx2.at[j]], sem, add=True)
        return 0

    lax.fori_loop(0, DEG_WIN, prime, 0)

    def step(j, _):
        pltpu.make_async_copy(ones_src, deg_sh.at[idx2.at[j]], sem).wait()

        @pl.when(j < NCHS - DEG_WIN)
        def _():
            pltpu.async_copy(ones_src, deg_sh.at[idx2.at[j + DEG_WIN]], sem,
                             add=True)

        return 0

    lax.fori_loop(0, NCHS, step, 0)

    plsc.subcore_barrier()

    @pl.when(sid == 0)
    def _():
        pltpu.sync_copy(deg_sh, out_hbm.at[cid])


@functools.partial(
    pl.kernel,
    out_type=jax.ShapeDtypeStruct((NC, N, D), jnp.float32),
    mesh=_mesh,
    scratch_types=[
        pltpu.VMEM((NCHS, CHS), jnp.int32),
        [pltpu.VMEM((GRP, CHS), jnp.int32)] * 2,
        [pltpu.VMEM((CHS, D), jnp.float32)] * 2,
        [pltpu.SemaphoreType.DMA] * 2,
        [pltpu.SemaphoreType.DMA] * 2,
        [pltpu.SemaphoreType.DMA] * 2,
        pltpu.SemaphoreType.DMA,
        pltpu.VMEM_SHARED((N_ACC, D), jnp.float32),
    ],
)
def _spmm(rows_hbm, cols_hbm, g_hbm, acc_hbm,
          colv2, rowbs, gbufs, gsems, ssems, rsems, isem, acc_sh):
    cid = lax.axis_index("c")
    sid = lax.axis_index("s")
    wid = sid * NC + cid
    ngrp = NCHS // GRP

    def rows_src(g):
        return rows_hbm.at[wid, pl.ds(pl.multiple_of(g * GRP, GRP), GRP)]

    c_idx = pltpu.async_copy(cols_hbm.at[wid], colv2, isem)
    for p in range(2):
        pltpu.async_copy(rows_src(p), rowbs[p], rsems[p])

    r0 = sid * (N // NS)

    @pl.when(cid == 0)
    def _():
        @pl.when(sid < NS - 1)
        def _():
            s0 = sid * 624
            pltpu.sync_copy(g_hbm.at[pl.ds(s0, 624)],
                            acc_sh.at[pl.ds(s0, 624)])

        @pl.when(sid == NS - 1)
        def _():
            pltpu.sync_copy(g_hbm.at[pl.ds(624 * (NS - 1), 640)],
                            acc_sh.at[pl.ds(624 * (NS - 1), 640)])

    @pl.when(cid == 1)
    def _():
        zb = gbufs[0]

        def zrow(i, _):
            def zcol(j, _):
                zb[i, pl.ds(j * 16, 16)] = jnp.zeros((16,), jnp.float32)
                return 0
            lax.fori_loop(0, D // 16, zcol, 0)
            return 0

        lax.fori_loop(0, CHS, zrow, 0)
        for k in range(5):
            pltpu.sync_copy(zb, acc_sh.at[pl.ds(r0 + k * CHS, CHS)])

    c_idx.wait()
    plsc.subcore_barrier()

    for b in range(2):
        pltpu.async_copy(g_hbm.at[colv2.at[b]], gbufs[b], gsems[b])

    def group(g, p):
        pltpu.make_async_copy(rows_src(g), rowbs[p], rsems[p]).wait()
        for i in range(GRP):
            b = i % 2
            j = g * GRP + i
            pltpu.make_async_copy(g_hbm.at[colv2.at[j]], gbufs[b],
                                  gsems[b]).wait()
            pltpu.sync_copy(gbufs[b], acc_sh.at[rowbs[p].at[i]], add=True)

            @pl.when(j < NCHS - 2)
            def _(j=j, b=b):
                pltpu.async_copy(g_hbm.at[colv2.at[j + 2]], gbufs[b],
                                 gsems[b])

        @pl.when(g < ngrp - 2)
        def _():
            pltpu.async_copy(rows_src(g + 2), rowbs[p], rsems[p])

    def pair(kk, _):
        group(2 * kk, 0)
        group(2 * kk + 1, 1)
        return 0

    lax.fori_loop(0, ngrp // 2, pair, 0)

    plsc.subcore_barrier()

    @pl.when(sid < NS - 1)
    def _():
        s0 = sid * 624
        pltpu.sync_copy(acc_sh.at[pl.ds(s0, 624)],
                        acc_hbm.at[cid, pl.ds(s0, 624)])

    @pl.when(sid == NS - 1)
    def _():
        pltpu.sync_copy(acc_sh.at[pl.ds(624 * (NS - 1), 640)],
                        acc_hbm.at[cid, pl.ds(624 * (NS - 1), 640)])


def _linear_body(x_ref, wt_ref, b_ref, deg_ref, g_ref):
    h = jnp.dot(x_ref[...], wt_ref[...], preferred_element_type=jnp.float32)
    h = h + b_ref[...]
    deg = jnp.sum(deg_ref[...], axis=1, keepdims=True) + 1.0
    g_ref[...] = h * lax.rsqrt(deg)


def _combine_body(acc_ref, deg_ref, o_ref):
    deg = jnp.sum(deg_ref[...], axis=1, keepdims=True) + 1.0
    o_ref[...] = (acc_ref[0] + acc_ref[1]) * lax.rsqrt(deg)


def kernel(x, edge_index, W, b):
    ei = edge_index.astype(jnp.int32)
    npad = E_PAD - E
    pad_r = N + (jnp.arange(npad, dtype=jnp.int32) % (N_ACC - N))
    pad_c = jnp.arange(npad, dtype=jnp.int32) % N
    rows3 = jnp.concatenate([ei[0], pad_r]).reshape(NW, NCHS, CHS)
    cols3 = jnp.concatenate([ei[1], pad_c]).reshape(NW, NCHS, CHS)

    degs = _deg(rows3)
    degs_t = degs[:, :N].T

    g = pl.pallas_call(
        _linear_body,
        out_shape=jax.ShapeDtypeStruct((N, D), jnp.float32),
    )(x, W.T, b.reshape(1, D), degs_t)

    accs = _spmm(rows3, cols3, g)

    out = pl.pallas_call(
        _combine_body,
        out_shape=jax.ShapeDtypeStruct((N, D), jnp.float32),
    )(accs, degs_t)
    return out


# speedup vs baseline: 1.2530x; 1.0354x over previous
import functools

import jax
import jax.numpy as jnp
from jax import lax
from jax.experimental import pallas as pl
from jax.experimental.pallas import tpu as pltpu
from jax.experimental.pallas import tpu_sc as plsc

N = 10000
E = 320000
D = 128

NC = 2
NS = 16
NW = NC * NS

CHS = 128
NCHS = 80
E_PAD = NW * NCHS * CHS
E_W = E_PAD // NW
GRP = 8

N_PAD = 10240
N_ACC = 10080
DEG_WIN = 16

_mesh = plsc.VectorSubcoreMesh(
    core_axis_name="c", subcore_axis_name="s", num_cores=NC, num_subcores=NS
)


def _fill_1d(ref, n, value):
    v = jnp.full((16,), value, jnp.float32)

    def body(i, _):
        ref[pl.ds(i * 16, 16)] = v
        return 0

    lax.fori_loop(0, n // 16, body, 0)


@functools.partial(
    pl.kernel,
    out_type=jax.ShapeDtypeStruct((NC, N_PAD), jnp.float32),
    mesh=_mesh,
    scratch_types=[
        pltpu.VMEM((NCHS, CHS), jnp.int32),
        pltpu.VMEM((128,), jnp.float32),
        pltpu.VMEM((640,), jnp.float32),
        pltpu.VMEM_SHARED((N_PAD,), jnp.float32),
        pltpu.SemaphoreType.DMA,
    ],
)
def _deg(ei_hbm, out_hbm, idx2, ones_v, zeros_v, deg_sh, sem):
    cid = lax.axis_index("c")
    sid = lax.axis_index("s")
    wid = sid * NC + cid

    _fill_1d(ones_v, 128, 1.0)
    _fill_1d(zeros_v, 640, 0.0)
    pltpu.sync_copy(zeros_v, deg_sh.at[pl.ds(sid * 640, 640)])
    pltpu.sync_copy(ei_hbm.at[0, wid], idx2)
    plsc.subcore_barrier()

    ones_src = ones_v.at[pl.ds(0, CHS)]

    def prime(j, _):
        pltpu.async_copy(ones_src, deg_sh.at[idx2.at[j]], sem, add=True)
        return 0

    lax.fori_loop(0, DEG_WIN, prime, 0)

    def step(j, _):
        pltpu.make_async_copy(ones_src, deg_sh.at[idx2.at[j]], sem).wait()

        @pl.when(j < NCHS - DEG_WIN)
        def _():
            pltpu.async_copy(ones_src, deg_sh.at[idx2.at[j + DEG_WIN]], sem,
                             add=True)

        return 0

    lax.fori_loop(0, NCHS, step, 0)

    plsc.subcore_barrier()

    @pl.when(sid == 0)
    def _():
        pltpu.sync_copy(deg_sh, out_hbm.at[cid])


@functools.partial(
    pl.kernel,
    out_type=jax.ShapeDtypeStruct((NC, N, D), jnp.float32),
    mesh=_mesh,
    scratch_types=[
        pltpu.VMEM((NCHS, CHS), jnp.int32),
        [pltpu.VMEM((GRP, CHS), jnp.int32)] * 2,
        [pltpu.VMEM((CHS, D), jnp.float32)] * 2,
        [pltpu.SemaphoreType.DMA] * 2,
        [pltpu.SemaphoreType.DMA] * 2,
        [pltpu.SemaphoreType.DMA] * 2,
        pltpu.SemaphoreType.DMA,
        pltpu.VMEM_SHARED((N_ACC, D), jnp.float32),
    ],
)
def _spmm(ei_hbm, g_hbm, acc_hbm,
          colv2, rowbs, gbufs, gsems, ssems, rsems, isem, acc_sh):
    cid = lax.axis_index("c")
    sid = lax.axis_index("s")
    wid = sid * NC + cid
    ngrp = NCHS // GRP

    def rows_src(g):
        return ei_hbm.at[0, wid, pl.ds(pl.multiple_of(g * GRP, GRP), GRP)]

    c_idx = pltpu.async_copy(ei_hbm.at[1, wid], colv2, isem)
    for p in range(2):
        pltpu.async_copy(rows_src(p), rowbs[p], rsems[p])

    r0 = sid * (N // NS)

    @pl.when(cid == 0)
    def _():
        @pl.when(sid < NS - 1)
        def _():
            s0 = sid * 624
            pltpu.sync_copy(g_hbm.at[pl.ds(s0, 624)],
                            acc_sh.at[pl.ds(s0, 624)])

        @pl.when(sid == NS - 1)
        def _():
            pltpu.sync_copy(g_hbm.at[pl.ds(624 * (NS - 1), 640)],
                            acc_sh.at[pl.ds(624 * (NS - 1), 640)])

    @pl.when(cid == 1)
    def _():
        zb = gbufs[0]

        def zrow(i, _):
            def zcol(j, _):
                zb[i, pl.ds(j * 16, 16)] = jnp.zeros((16,), jnp.float32)
                return 0
            lax.fori_loop(0, D // 16, zcol, 0)
            return 0

        lax.fori_loop(0, CHS, zrow, 0)
        for k in range(5):
            pltpu.sync_copy(zb, acc_sh.at[pl.ds(r0 + k * CHS, CHS)])

    c_idx.wait()
    plsc.subcore_barrier()

    for b in range(2):
        pltpu.async_copy(g_hbm.at[colv2.at[b]], gbufs[b], gsems[b])

    def group(g, p):
        pltpu.make_async_copy(rows_src(g), rowbs[p], rsems[p]).wait()
        for i in range(GRP):
            b = i % 2
            j = g * GRP + i
            pltpu.make_async_copy(g_hbm.at[colv2.at[j]], gbufs[b],
                                  gsems[b]).wait()
            pltpu.sync_copy(gbufs[b], acc_sh.at[rowbs[p].at[i]], add=True)

            @pl.when(j < NCHS - 2)
            def _(j=j, b=b):
                pltpu.async_copy(g_hbm.at[colv2.at[j + 2]], gbufs[b],
                                 gsems[b])

        @pl.when(g < ngrp - 2)
        def _():
            pltpu.async_copy(rows_src(g + 2), rowbs[p], rsems[p])

    def pair(kk, _):
        group(2 * kk, 0)
        group(2 * kk + 1, 1)
        return 0

    lax.fori_loop(0, ngrp // 2, pair, 0)

    plsc.subcore_barrier()

    @pl.when(sid < NS - 1)
    def _():
        s0 = sid * 624
        pltpu.sync_copy(acc_sh.at[pl.ds(s0, 624)],
                        acc_hbm.at[cid, pl.ds(s0, 624)])

    @pl.when(sid == NS - 1)
    def _():
        pltpu.sync_copy(acc_sh.at[pl.ds(624 * (NS - 1), 640)],
                        acc_hbm.at[cid, pl.ds(624 * (NS - 1), 640)])


def _linear_body(x_ref, wt_ref, b_ref, deg_ref, g_ref):
    h = jnp.dot(x_ref[...], wt_ref[...], preferred_element_type=jnp.float32)
    h = h + b_ref[...]
    deg = jnp.sum(deg_ref[...], axis=1, keepdims=True) + 1.0
    g_ref[...] = h * lax.rsqrt(deg)


def _combine_body(acc_ref, deg_ref, o_ref):
    deg = jnp.sum(deg_ref[...], axis=1, keepdims=True) + 1.0
    o_ref[...] = (acc_ref[0] + acc_ref[1]) * lax.rsqrt(deg)


def kernel(x, edge_index, W, b):
    ei = edge_index.astype(jnp.int32)
    npad = E_PAD - E
    pad_r = N + (jnp.arange(npad, dtype=jnp.int32) % (N_ACC - N))
    pad_c = jnp.arange(npad, dtype=jnp.int32) % N
    ei4 = jnp.concatenate([ei, jnp.stack([pad_r, pad_c])], axis=1)
    ei4 = ei4.reshape(2, NW, NCHS, CHS)

    degs = _deg(ei4)
    degs_t = degs[:, :N].T

    g = pl.pallas_call(
        _linear_body,
        out_shape=jax.ShapeDtypeStruct((N, D), jnp.float32),
    )(x, W.T, b.reshape(1, D), degs_t)

    accs = _spmm(ei4, g)

    out = pl.pallas_call(
        _combine_body,
        out_shape=jax.ShapeDtypeStruct((N, D), jnp.float32),
    )(accs, degs_t)
    return out


# speedup vs baseline: 1.2662x; 1.0105x over previous
import functools

import jax
import jax.numpy as jnp
import numpy as np
from jax import lax
from jax.experimental import pallas as pl
from jax.experimental.pallas import tpu as pltpu
from jax.experimental.pallas import tpu_sc as plsc

N = 10000
E = 320000
D = 128

NC = 2
NS = 16
NW = NC * NS

CHS = 128
NCHS = 80
E_PAD = NW * NCHS * CHS
E_W = E_PAD // NW
GRP = 8

N_PAD = 10240
N_ACC = 10080
DEG_WIN = 16

_mesh = plsc.VectorSubcoreMesh(
    core_axis_name="c", subcore_axis_name="s", num_cores=NC, num_subcores=NS
)


def _fill_1d(ref, n, value):
    v = jnp.full((16,), value, jnp.float32)

    def body(i, _):
        ref[pl.ds(i * 16, 16)] = v
        return 0

    lax.fori_loop(0, n // 16, body, 0)


@functools.partial(
    pl.kernel,
    out_type=jax.ShapeDtypeStruct((NC, N_PAD), jnp.float32),
    mesh=_mesh,
    scratch_types=[
        pltpu.VMEM((NCHS, CHS), jnp.int32),
        pltpu.VMEM((128,), jnp.float32),
        pltpu.VMEM((640,), jnp.float32),
        pltpu.VMEM_SHARED((N_PAD,), jnp.float32),
        pltpu.SemaphoreType.DMA,
    ],
)
def _deg(ei_hbm, out_hbm, idx2, ones_v, zeros_v, deg_sh, sem):
    cid = lax.axis_index("c")
    sid = lax.axis_index("s")
    wid = sid * NC + cid

    _fill_1d(ones_v, 128, 1.0)
    _fill_1d(zeros_v, 640, 0.0)
    pltpu.sync_copy(zeros_v, deg_sh.at[pl.ds(sid * 640, 640)])
    pltpu.sync_copy(ei_hbm.at[0, wid], idx2)
    plsc.subcore_barrier()

    ones_src = ones_v.at[pl.ds(0, CHS)]

    def prime(j, _):
        pltpu.async_copy(ones_src, deg_sh.at[idx2.at[j]], sem, add=True)
        return 0

    lax.fori_loop(0, DEG_WIN, prime, 0)

    def step(j, _):
        pltpu.make_async_copy(ones_src, deg_sh.at[idx2.at[j]], sem).wait()

        @pl.when(j < NCHS - DEG_WIN)
        def _():
            pltpu.async_copy(ones_src, deg_sh.at[idx2.at[j + DEG_WIN]], sem,
                             add=True)

        return 0

    lax.fori_loop(0, NCHS, step, 0)

    plsc.subcore_barrier()

    @pl.when(sid == 0)
    def _():
        pltpu.sync_copy(deg_sh, out_hbm.at[cid])


@functools.partial(
    pl.kernel,
    out_type=jax.ShapeDtypeStruct((NC, N, D), jnp.float32),
    mesh=_mesh,
    scratch_types=[
        pltpu.VMEM((NCHS, CHS), jnp.int32),
        [pltpu.VMEM((GRP, CHS), jnp.int32)] * 2,
        [pltpu.VMEM((CHS, D), jnp.float32)] * 2,
        [pltpu.SemaphoreType.DMA] * 2,
        [pltpu.SemaphoreType.DMA] * 2,
        [pltpu.SemaphoreType.DMA] * 2,
        pltpu.SemaphoreType.DMA,
        pltpu.VMEM_SHARED((N_ACC, D), jnp.float32),
    ],
)
def _spmm(ei_hbm, g_hbm, acc_hbm,
          colv2, rowbs, gbufs, gsems, ssems, rsems, isem, acc_sh):
    cid = lax.axis_index("c")
    sid = lax.axis_index("s")
    wid = sid * NC + cid
    ngrp = NCHS // GRP

    def rows_src(g):
        return ei_hbm.at[0, wid, pl.ds(pl.multiple_of(g * GRP, GRP), GRP)]

    c_idx = pltpu.async_copy(ei_hbm.at[1, wid], colv2, isem)
    for p in range(2):
        pltpu.async_copy(rows_src(p), rowbs[p], rsems[p])

    r0 = sid * (N // NS)

    @pl.when(cid == 0)
    def _():
        @pl.when(sid < NS - 1)
        def _():
            s0 = sid * 624
            pltpu.sync_copy(g_hbm.at[pl.ds(s0, 624)],
                            acc_sh.at[pl.ds(s0, 624)])

        @pl.when(sid == NS - 1)
        def _():
            pltpu.sync_copy(g_hbm.at[pl.ds(624 * (NS - 1), 640)],
                            acc_sh.at[pl.ds(624 * (NS - 1), 640)])

    @pl.when(cid == 1)
    def _():
        zb = gbufs[0]

        def zrow(i, _):
            def zcol(j, _):
                zb[i, pl.ds(j * 16, 16)] = jnp.zeros((16,), jnp.float32)
                return 0
            lax.fori_loop(0, D // 16, zcol, 0)
            return 0

        lax.fori_loop(0, CHS, zrow, 0)
        for k in range(5):
            pltpu.sync_copy(zb, acc_sh.at[pl.ds(r0 + k * CHS, CHS)])

    c_idx.wait()
    plsc.subcore_barrier()

    for b in range(2):
        pltpu.async_copy(g_hbm.at[colv2.at[b]], gbufs[b], gsems[b])

    def group(g, p):
        pltpu.make_async_copy(rows_src(g), rowbs[p], rsems[p]).wait()
        for i in range(GRP):
            b = i % 2
            j = g * GRP + i
            pltpu.make_async_copy(g_hbm.at[colv2.at[j]], gbufs[b],
                                  gsems[b]).wait()
            pltpu.sync_copy(gbufs[b], acc_sh.at[rowbs[p].at[i]], add=True)

            @pl.when(j < NCHS - 2)
            def _(j=j, b=b):
                pltpu.async_copy(g_hbm.at[colv2.at[j + 2]], gbufs[b],
                                 gsems[b])

        @pl.when(g < ngrp - 2)
        def _():
            pltpu.async_copy(rows_src(g + 2), rowbs[p], rsems[p])

    def pair(kk, _):
        group(2 * kk, 0)
        group(2 * kk + 1, 1)
        return 0

    lax.fori_loop(0, ngrp // 2, pair, 0)

    plsc.subcore_barrier()

    @pl.when(sid < NS - 1)
    def _():
        s0 = sid * 624
        pltpu.sync_copy(acc_sh.at[pl.ds(s0, 624)],
                        acc_hbm.at[cid, pl.ds(s0, 624)])

    @pl.when(sid == NS - 1)
    def _():
        pltpu.sync_copy(acc_sh.at[pl.ds(624 * (NS - 1), 640)],
                        acc_hbm.at[cid, pl.ds(624 * (NS - 1), 640)])


def _linear_body(x_ref, wt_ref, b_ref, deg_ref, g_ref):
    h = jnp.dot(x_ref[...], wt_ref[...], preferred_element_type=jnp.float32)
    h = h + b_ref[...]
    deg = jnp.sum(deg_ref[...], axis=1, keepdims=True) + 1.0
    g_ref[...] = h * lax.rsqrt(deg)


def _combine_body(acc_ref, deg_ref, o_ref):
    deg = jnp.sum(deg_ref[...], axis=1, keepdims=True) + 1.0
    o_ref[...] = (acc_ref[0] + acc_ref[1]) * lax.rsqrt(deg)


def kernel(x, edge_index, W, b):
    ei = edge_index.astype(jnp.int32)
    npad = E_PAD - E
    pad_np = np.stack([N + (np.arange(npad) % (N_ACC - N)),
                       np.arange(npad) % N]).astype(np.int32)
    ei4 = jnp.concatenate([ei, jnp.asarray(pad_np)], axis=1)
    ei4 = ei4.reshape(2, NW, NCHS, CHS)

    degs = _deg(ei4)
    degs_t = degs[:, :N].T

    g = pl.pallas_call(
        _linear_body,
        grid=(5,),
        in_specs=[
            pl.BlockSpec((N // 5, D), lambda i: (i, 0)),
            pl.BlockSpec((D, D), lambda i: (0, 0)),
            pl.BlockSpec((1, D), lambda i: (0, 0)),
            pl.BlockSpec((N // 5, 2), lambda i: (i, 0)),
        ],
        out_specs=pl.BlockSpec((N // 5, D), lambda i: (i, 0)),
        out_shape=jax.ShapeDtypeStruct((N, D), jnp.float32),
    )(x, W.T, b.reshape(1, D), degs_t)

    accs = _spmm(ei4, g)

    out = pl.pallas_call(
        _combine_body,
        grid=(5,),
        in_specs=[
            pl.BlockSpec((NC, N // 5, D), lambda i: (0, i, 0)),
            pl.BlockSpec((N // 5, 2), lambda i: (i, 0)),
        ],
        out_specs=pl.BlockSpec((N // 5, D), lambda i: (i, 0)),
        out_shape=jax.ShapeDtypeStruct((N, D), jnp.float32),
    )(accs, degs_t)
    return out
